# baseline (speedup 1.0000x reference)
import functools

import jax
import jax.numpy as jnp
from jax import lax
from jax.experimental import pallas as pl
from jax.experimental.pallas import tpu as pltpu
from jax.experimental.pallas import tpu_sc as plsc

L = 16
GS = 4
NIB = 3


def _sc_body(B, C, D, H, W, x_hbm, dt_hbm, ht_hbm, wt_hbm, out_hbm,
             htc, wtc, dtc, hsbuf, ib0, ib1, ib2, ob0, ob1,
             si0, si1, si2, so0, so1):
    NC = 2
    NS = 16
    NW = NC * NS
    planes = B * C * D
    per_w = planes // NW
    cc_n = per_w // D
    HW = H * W
    NG = D // GS
    HALF = GS * HW // 2

    wid = lax.axis_index("s") * NC + lax.axis_index("c")
    p0 = wid * per_w

    ib = [ib0, ib1, ib2]
    ob = [ob0, ob1]
    sin = [si0, si1, si2]
    sout = [so0, so1]

    def cc_loop(cc, _):
        plane0 = p0 + cc * D
        c = (plane0 // D) % C
        pltpu.sync_copy(ht_hbm.at[c], htc)
        pltpu.sync_copy(wt_hbm.at[c], wtc)
        pltpu.sync_copy(dt_hbm.at[c], dtc)

        wtv = [wtc[pl.ds(wv * L, L)] for wv in range(W // L)]
        dtv = [dtc[pl.ds(k * L, L)] for k in range(D // L)]

        for hv in range(H // L):
            hvec = htc[pl.ds(hv * L, L)]
            for li in range(L):
                h = hv * L + li
                hsbuf[pl.ds(h * L, L)] = jnp.full((L,), hvec[li], jnp.float32)

        def start_in(g, slot):
            off = (plane0 + g * GS) * HW
            return [
                pltpu.async_copy(x_hbm.at[pl.ds(off, HALF)],
                                 ib[slot].at[pl.ds(0, HALF)], sin[slot]),
                pltpu.async_copy(x_hbm.at[pl.ds(off + HALF, HALF)],
                                 ib[slot].at[pl.ds(HALF, HALF)], sin[slot]),
            ]

        def start_out(g, slot):
            off = (plane0 + g * GS) * HW
            return [
                pltpu.async_copy(ob[slot].at[pl.ds(0, HALF)],
                                 out_hbm.at[pl.ds(off, HALF)], sout[slot]),
                pltpu.async_copy(ob[slot].at[pl.ds(HALF, HALF)],
                                 out_hbm.at[pl.ds(off + HALF, HALF)], sout[slot]),
            ]

        in_desc = {0: start_in(0, 0), 1: start_in(1, 1)}
        out_desc = {}
        for g in range(NG):
            islot = g % NIB
            oslot = g & 1
            if g + 2 < NG:
                in_desc[g + 2] = start_in(g + 2, (g + 2) % NIB)
            for dd in in_desc.pop(g):
                dd.wait()
            if g >= 2:
                for dd in out_desc.pop(g - 2):
                    dd.wait()
            ibuf, obuf = ib[islot], ob[oslot]
            for dl in range(GS):
                d = g * GS + dl
                sv = jnp.full((L,), dtv[d // L][d % L], jnp.float32)

                @plsc.parallel_loop(0, H, step=1, unroll=8)
                def row_loop(h, dl=dl, sv=sv, ibuf=ibuf, obuf=obuf):
                    hs = hsbuf[pl.ds(h * L, L)] + sv
                    base = dl * HW + h * W
                    for wv in range(W // L):
                        sl = pl.ds(base + wv * L, L)
                        obuf[sl] = ibuf[sl] + (hs + wtv[wv])

            out_desc[g] = start_out(g, oslot)
        for g in (NG - 2, NG - 1):
            for dd in out_desc.pop(g):
                dd.wait()
        return 0

    lax.fori_loop(0, cc_n, cc_loop, 0)


@jax.jit
def kernel(x, depth_table, height_table, width_table):
    B, C, D, H, W = x.shape
    N = B * C * D * H * W
    xf = x.reshape(N)
    dt_t = depth_table.T
    ht_t = height_table.T
    wt_t = width_table.T

    mesh = plsc.VectorSubcoreMesh(core_axis_name="c", subcore_axis_name="s")
    body = functools.partial(_sc_body, B, C, D, H, W)
    out = pl.kernel(
        body,
        out_type=jax.ShapeDtypeStruct((N,), jnp.float32),
        mesh=mesh,
        scratch_types=[
            pltpu.VMEM((H,), jnp.float32),
            pltpu.VMEM((W,), jnp.float32),
            pltpu.VMEM((D,), jnp.float32),
            pltpu.VMEM((H * L,), jnp.float32),
            pltpu.VMEM((GS * H * W,), jnp.float32),
            pltpu.VMEM((GS * H * W,), jnp.float32),
            pltpu.VMEM((GS * H * W,), jnp.float32),
            pltpu.VMEM((GS * H * W,), jnp.float32),
            pltpu.VMEM((GS * H * W,), jnp.float32),
            pltpu.SemaphoreType.DMA,
            pltpu.SemaphoreType.DMA,
            pltpu.SemaphoreType.DMA,
            pltpu.SemaphoreType.DMA,
            pltpu.SemaphoreType.DMA,
        ],
    )(xf, dt_t, ht_t, wt_t)
    return out.reshape(B, C, D, H, W)
